# baseline (speedup 1.0000x reference)
import functools

import jax
import jax.numpy as jnp
from jax import lax
from jax.experimental import pallas as pl
from jax.experimental.pallas import tpu as pltpu
from jax.experimental.pallas import tpu_sc as plsc

NC = 2
NS = 16
NW = NC * NS
D = 64
VOCAB = 1000000
HALF = VOCAB // 2
BATCH = 16384
HIST = 20
BPW = BATCH // NW
CH = 256
NCHUNK = HIST * 2
IPW = BPW * HIST
BJ = 8192


def _pack_table(tT):
    def body(l_ref, r_ref, o_ref):
        eye = jnp.eye(D, dtype=jnp.float32)
        dn = (((0,), (0,)), ((), ()))
        o_ref[:, 0:D] = lax.dot_general(
            l_ref[...], eye, dn, preferred_element_type=jnp.float32
        )
        o_ref[:, D:2 * D] = lax.dot_general(
            r_ref[...], eye, dn, preferred_element_type=jnp.float32
        )

    nb = (HALF + BJ - 1) // BJ
    last = VOCAB // BJ - 1
    return pl.pallas_call(
        body,
        grid=(nb,),
        in_specs=[
            pl.BlockSpec((D, BJ), lambda j: (0, 2 * j)),
            pl.BlockSpec(
                (D, BJ), lambda j: (0, jnp.minimum(2 * j + 1, last))
            ),
        ],
        out_specs=pl.BlockSpec((BJ, 2 * D), lambda j: (j, 0)),
        out_shape=jax.ShapeDtypeStruct((nb * BJ, 2 * D), jnp.float32),
        compiler_params=pltpu.CompilerParams(
            fuse_transposed_lhs_in_matmul=True,
            dimension_semantics=("arbitrary",),
        ),
    )(tT, tT)


def _embed(idsp, tab):
    mesh = plsc.VectorSubcoreMesh(core_axis_name="c", subcore_axis_name="s")

    @functools.partial(
        pl.kernel,
        out_type=jax.ShapeDtypeStruct((HIST, D, BATCH), jnp.float32),
        mesh=mesh,
        scratch_types=[
            pltpu.VMEM((IPW,), jnp.int32),
            pltpu.VMEM((2, CH, D), jnp.float32),
            pltpu.VMEM((2, D, CH + 1), jnp.float32),
            pltpu.SemaphoreType.DMA,
            pltpu.SemaphoreType.DMA,
        ],
        compiler_params=pltpu.CompilerParams(
            use_tc_tiling_on_sc=False, needs_layout_passes=False
        ),
    )
    def k(ids_hbm, tab_hbm, out_hbm, idx_v, gbuf, tbuf, gsem, wsem):
        wid = lax.axis_index("s") * NC + lax.axis_index("c")
        b0 = wid * BPW
        pltpu.sync_copy(ids_hbm.at[pl.ds(wid * IPW, IPW)], idx_v)
        lane = lax.iota(jnp.int32, 16)

        def gather_of(t, buf):
            return pltpu.make_async_copy(
                tab_hbm.at[idx_v.at[pl.ds(t * CH, CH)]], gbuf.at[buf], gsem
            )

        def write_of(t, buf):
            h = t // 2
            c0 = (t % 2) * CH
            return pltpu.make_async_copy(
                tbuf.at[buf].at[:, pl.ds(0, CH)],
                out_hbm.at[h].at[:, pl.ds(b0 + c0, CH)],
                wsem,
            )

        gather_of(0, 0).start()

        def chunk(t, _):
            cur = lax.rem(t, 2)
            nxt = lax.rem(t + 1, 2)

            @pl.when(t + 1 < NCHUNK)
            def _():
                gather_of(t + 1, nxt).start()

            gather_of(t, cur).wait()

            @pl.when(t >= 2)
            def _():
                write_of(t - 2, cur).wait()

            g = gbuf.at[cur]
            tb = tbuf.at[cur]
            dvecs = [16 * j + lane for j in range(D // 16)]
            for k in range(CH):
                kvec = jnp.full((16,), k, jnp.int32)
                xs = [
                    plsc.load_gather(g, [kvec, dv]) for dv in dvecs
                ]
                for dv, x in zip(dvecs, xs):
                    plsc.store_scatter(tb, [dv, kvec], x)

            write_of(t, cur).start()
            return ()

        lax.fori_loop(0, NCHUNK, chunk, (), unroll=False)
        write_of(NCHUNK - 2, 0).wait()
        write_of(NCHUNK - 1, 1).wait()

    return k(idsp, tab)


def kernel(ids, embeddings):
    t2 = _pack_table(embeddings.T)
    lb = BJ.bit_length() - 1
    rows = (
        ((ids >> (lb + 1)) << (lb + 1))
        + ((ids & (BJ - 1)) << 1)
        + ((ids >> lb) & 1)
    )
    idsp = rows.T.reshape(HIST, NW, BPW).transpose(1, 0, 2).reshape(-1)
    out = _embed(idsp, t2.reshape(t2.shape[0] * 2, D))
    return jnp.transpose(out, (2, 0, 1))


# speedup vs baseline: 1.0065x; 1.0065x over previous
import functools

import jax
import jax.numpy as jnp
from jax import lax
from jax.experimental import pallas as pl
from jax.experimental.pallas import tpu as pltpu
from jax.experimental.pallas import tpu_sc as plsc

NC = 2
NS = 16
NW = NC * NS
D = 64
VOCAB = 1000000
HALF = VOCAB // 2
BATCH = 16384
HIST = 20
BPW = BATCH // NW
CH = 256
NCHUNK = HIST * 2
IPW = BPW * HIST
BJ = 8192


def _pack_table(tT):
    def body(l_ref, r_ref, o_ref):
        eye = jnp.eye(D, dtype=jnp.float32)
        dn = (((0,), (0,)), ((), ()))
        o_ref[:, 0:D] = lax.dot_general(
            l_ref[...], eye, dn, preferred_element_type=jnp.float32
        )
        o_ref[:, D:2 * D] = lax.dot_general(
            r_ref[...], eye, dn, preferred_element_type=jnp.float32
        )

    nb = (HALF + BJ - 1) // BJ
    last = VOCAB // BJ - 1
    return pl.pallas_call(
        body,
        grid=(nb,),
        in_specs=[
            pl.BlockSpec((D, BJ), lambda j: (0, 2 * j)),
            pl.BlockSpec(
                (D, BJ), lambda j: (0, jnp.minimum(2 * j + 1, last))
            ),
        ],
        out_specs=pl.BlockSpec((BJ, 2 * D), lambda j: (j, 0)),
        out_shape=jax.ShapeDtypeStruct((nb * BJ, 2 * D), jnp.float32),
        compiler_params=pltpu.CompilerParams(
            fuse_transposed_lhs_in_matmul=True,
            dimension_semantics=("arbitrary",),
        ),
    )(tT, tT)


def _embed(idsp, tab):
    mesh = plsc.VectorSubcoreMesh(core_axis_name="c", subcore_axis_name="s")

    @functools.partial(
        pl.kernel,
        out_type=jax.ShapeDtypeStruct((HIST * D, BATCH), jnp.float32),
        mesh=mesh,
        scratch_types=[
            pltpu.VMEM((IPW,), jnp.int32),
            pltpu.VMEM((3, CH, D), jnp.float32),
            pltpu.VMEM((2, D, CH + 1), jnp.float32),
            pltpu.SemaphoreType.DMA,
            pltpu.SemaphoreType.DMA,
        ],
        compiler_params=pltpu.CompilerParams(
            use_tc_tiling_on_sc=False, needs_layout_passes=False
        ),
    )
    def k(ids_hbm, tab_hbm, out_hbm, idx_v, gbuf, tbuf, gsem, wsem):
        wid = lax.axis_index("s") * NC + lax.axis_index("c")
        b0 = wid * BPW
        pltpu.sync_copy(ids_hbm.at[pl.ds(wid * IPW, IPW)], idx_v)
        lane = lax.iota(jnp.int32, 16)

        def gather_of(t, buf):
            return pltpu.make_async_copy(
                tab_hbm.at[idx_v.at[pl.ds(t * CH, CH)]], gbuf.at[buf], gsem
            )

        def write_of(t, buf):
            h = t // 2
            c0 = (t % 2) * CH
            return pltpu.make_async_copy(
                tbuf.at[buf].at[:, pl.ds(0, CH)],
                out_hbm.at[pl.ds(h * D, D)].at[:, pl.ds(b0 + c0, CH)],
                wsem,
            )

        gather_of(0, 0).start()
        gather_of(1, 1).start()

        def chunk(t, _):
            cur = lax.rem(t, 2)
            gcur = lax.rem(t, 3)

            @pl.when(t + 2 < NCHUNK)
            def _():
                gather_of(t + 2, lax.rem(t + 2, 3)).start()

            gather_of(t, gcur).wait()

            @pl.when(t >= 2)
            def _():
                write_of(t - 2, cur).wait()

            g = gbuf.at[gcur]
            tb = tbuf.at[cur]
            dvecs = [16 * j + lane for j in range(D // 16)]
            for k in range(CH):
                kvec = jnp.full((16,), k, jnp.int32)
                xs = [
                    plsc.load_gather(g, [kvec, dv]) for dv in dvecs
                ]
                for dv, x in zip(dvecs, xs):
                    plsc.store_scatter(tb, [dv, kvec], x)

            write_of(t, cur).start()
            return ()

        lax.fori_loop(0, NCHUNK, chunk, (), unroll=False)
        write_of(NCHUNK - 2, 0).wait()
        write_of(NCHUNK - 1, 1).wait()

    return k(idsp, tab)


def kernel(ids, embeddings):
    t2 = _pack_table(embeddings.T)
    lb = BJ.bit_length() - 1
    rows = (
        ((ids >> (lb + 1)) << (lb + 1))
        + ((ids & (BJ - 1)) << 1)
        + ((ids >> lb) & 1)
    )
    idsp = rows.T.reshape(HIST, NW, BPW).transpose(1, 0, 2).reshape(-1)
    out = _embed(idsp, t2.reshape(t2.shape[0] * 2, D))
    return jnp.transpose(out.reshape(HIST, D, BATCH), (2, 0, 1))
